# baseline (speedup 1.0000x reference)
import functools

import jax
import jax.numpy as jnp
from jax import lax
from jax.experimental import pallas as pl
from jax.experimental.pallas import tpu as pltpu
from jax.experimental.pallas import tpu_sc as plsc

_N_SRC = 4
_SIZE = 262144
_DELAY = 32
_BUF_LEN = _DELAY + 1

_LANE = 128
_NCB = _SIZE // _LANE

_info = plsc.get_sparse_core_info()
_NC, _NS, _NL = _info.num_cores, _info.num_subcores, _info.num_lanes
_NW = _NC * _NS
_CBW = _NCB // _NW
_HALF = _CBW // 2


_NSTAGE = 4
_STEP = _CBW // _NSTAGE


def _sc_body(ptr_hbm, buf_hbm, out_hbm, idx_v, *rest):
    bufs, sems = rest[:_NSTAGE], rest[_NSTAGE:]
    wid = lax.axis_index("s") * _NC + lax.axis_index("c")
    base = wid * _CBW
    pltpu.sync_copy(ptr_hbm, idx_v.at[pl.ds(0, 1)])
    ptr = idx_v[...][0]
    slot = (ptr + 2) % _BUF_LEN
    rds = [
        pltpu.async_copy(
            buf_hbm.at[slot, pl.ds(base + i * _STEP, _STEP)], bufs[i], sems[i])
        for i in range(_NSTAGE)
    ]
    wrs = []
    for i in range(_NSTAGE):
        rds[i].wait()
        wrs.append(pltpu.async_copy(
            bufs[i], out_hbm.at[pl.ds(base + i * _STEP, _STEP)], sems[i]))
    for w in wrs:
        w.wait()


_sc_slice = functools.partial(
    pl.kernel,
    out_type=jax.ShapeDtypeStruct((_NCB, _N_SRC, _LANE), jnp.float32),
    mesh=plsc.VectorSubcoreMesh(core_axis_name="c", subcore_axis_name="s"),
    scratch_types=[
        pltpu.VMEM((_NL,), jnp.int32),
        *([pltpu.VMEM((_STEP, _N_SRC, _LANE), jnp.float32)] * _NSTAGE),
        *([pltpu.SemaphoreType.DMA] * _NSTAGE),
    ],
)(_sc_body)


def kernel(spikes, buffer, ptr):
    del spikes
    ptr_arr = jnp.asarray(ptr, jnp.int32).reshape(1)
    buf4 = buffer.reshape(_N_SRC, _BUF_LEN, _NCB, _LANE).transpose(1, 2, 0, 3)
    out4 = _sc_slice(ptr_arr, buf4)
    return out4.transpose(1, 0, 2).reshape(_N_SRC, _SIZE)


# speedup vs baseline: 1.0124x; 1.0124x over previous
import functools

import jax
import jax.numpy as jnp
from jax import lax
from jax.experimental import pallas as pl
from jax.experimental.pallas import tpu as pltpu
from jax.experimental.pallas import tpu_sc as plsc

_N_SRC = 4
_SIZE = 262144
_DELAY = 32
_BUF_LEN = _DELAY + 1

_LANE = 128
_NCB = _SIZE // _LANE

_info = plsc.get_sparse_core_info()
_NC, _NS, _NL = _info.num_cores, _info.num_subcores, _info.num_lanes
_NW = _NC * _NS
_CBW = _NCB // _NW
_HALF = _CBW // 2


def _sc_body(ptr_hbm, buf_hbm, out_hbm, idx_v, a_v, b_v, sem_a, sem_b):
    wid = lax.axis_index("s") * _NC + lax.axis_index("c")
    base = wid * _CBW
    pltpu.sync_copy(ptr_hbm, idx_v.at[pl.ds(0, 1)])
    ptr = idx_v[...][0]
    slot = (ptr + 2) % _BUF_LEN
    rd_a = pltpu.async_copy(buf_hbm.at[slot, pl.ds(base, _HALF)], a_v, sem_a)
    rd_b = pltpu.async_copy(
        buf_hbm.at[slot, pl.ds(base + _HALF, _HALF)], b_v, sem_b)
    rd_a.wait()
    wr_a = pltpu.async_copy(a_v, out_hbm.at[pl.ds(base, _HALF)], sem_a)
    rd_b.wait()
    wr_b = pltpu.async_copy(b_v, out_hbm.at[pl.ds(base + _HALF, _HALF)], sem_b)
    wr_a.wait()
    wr_b.wait()


_sc_slice = functools.partial(
    pl.kernel,
    out_type=jax.ShapeDtypeStruct((_NCB, _N_SRC, _LANE), jnp.float32),
    mesh=plsc.VectorSubcoreMesh(core_axis_name="c", subcore_axis_name="s"),
    scratch_types=[
        pltpu.VMEM((_NL,), jnp.int32),
        pltpu.VMEM((_HALF, _N_SRC, _LANE), jnp.float32),
        pltpu.VMEM((_HALF, _N_SRC, _LANE), jnp.float32),
        pltpu.SemaphoreType.DMA,
        pltpu.SemaphoreType.DMA,
    ],
)(_sc_body)


def kernel(spikes, buffer, ptr):
    del spikes
    ptr_arr = jnp.asarray(ptr, jnp.int32).reshape(1)
    buf4 = buffer.reshape(_N_SRC, _BUF_LEN, _NCB, _LANE).transpose(1, 2, 0, 3)
    out4 = _sc_slice(ptr_arr, buf4)
    return out4.transpose(1, 0, 2).reshape(_N_SRC, _SIZE)
